# baseline (speedup 1.0000x reference)
import functools

import jax
import jax.numpy as jnp
from jax.experimental import pallas as pl
from jax.experimental.pallas import tpu as pltpu
from jax.experimental.pallas import tpu_sc as plsc

_ALPHA = 0.25
_IGNORE_INDEX = -100

_ROWS = 1024
_COLS = 100000
_RBLK = 256
_CBLK = 2048
_NCBLK = (_COLS + _CBLK - 1) // _CBLK

_LANES = 128
_TABLE_ROWS = _ROWS * _COLS // _LANES

_SC_INFO = plsc.get_sparse_core_info()
_NW = _SC_INFO.num_cores * _SC_INFO.num_subcores
_BPW = _ROWS // _NW

_sc_mesh = plsc.VectorSubcoreMesh(core_axis_name="c", subcore_axis_name="s")


@functools.partial(
    pl.kernel,
    mesh=_sc_mesh,
    out_type=jax.ShapeDtypeStruct((_ROWS, _LANES), jnp.float32),
    scratch_types=[
        pltpu.VMEM((_BPW,), jnp.int32),
        pltpu.VMEM((_BPW, _LANES), jnp.float32),
        pltpu.SemaphoreType.DMA,
    ],
)
def _sc_gather(table_hbm, idx_hbm, out_hbm, idx_v, rows_v, sem):
    wid = jax.lax.axis_index("s") * _SC_INFO.num_cores + jax.lax.axis_index("c")
    base = wid * _BPW
    pltpu.sync_copy(idx_hbm.at[pl.ds(base, _BPW)], idx_v)
    pltpu.async_copy(table_hbm.at[idx_v], rows_v, sem).wait()
    pltpu.sync_copy(rows_v, out_hbm.at[pl.ds(base, _BPW)])


def _focal_kernel(x_ref, tgt_ref, lane_ref, g_ref, out_ref, m_ref, s_ref):
    i = pl.program_id(0)
    j = pl.program_id(1)

    @pl.when(j == 0)
    def _init():
        m_ref[...] = jnp.full((_RBLK, 1), -jnp.inf, jnp.float32)
        s_ref[...] = jnp.zeros((_RBLK, 1), jnp.float32)

    x = x_ref[...]
    is_last = j == _NCBLK - 1

    @pl.when(jnp.logical_not(is_last))
    def _full_block():
        m_old = m_ref[...]
        m_new = jnp.maximum(m_old, jnp.max(x, axis=1, keepdims=True))
        s_ref[...] = s_ref[...] * jnp.exp(m_old - m_new) + jnp.sum(
            jnp.exp(x - m_new), axis=1, keepdims=True
        )
        m_ref[...] = m_new

    @pl.when(is_last)
    def _last_block():
        col = j * _CBLK + jax.lax.broadcasted_iota(
            jnp.int32, (_RBLK, _CBLK), 1
        )
        xm = jnp.where(col < _COLS, x, -jnp.inf)
        m_old = m_ref[...]
        m_new = jnp.maximum(m_old, jnp.max(xm, axis=1, keepdims=True))
        s = s_ref[...] * jnp.exp(m_old - m_new) + jnp.sum(
            jnp.exp(xm - m_new), axis=1, keepdims=True
        )
        lane_iota = jax.lax.broadcasted_iota(jnp.int32, (_RBLK, _LANES), 1)
        t = jnp.sum(
            jnp.where(lane_iota == lane_ref[...], g_ref[...], 0.0),
            axis=1,
            keepdims=True,
        )
        nll = m_new + jnp.log(s) - t
        tgt = tgt_ref[...]
        loss = jnp.where(tgt == _IGNORE_INDEX, 0.0, nll)
        pt = jnp.exp(-loss)
        fl = _ALPHA * (1.0 - pt) * (1.0 - pt) * loss
        partial = jnp.sum(fl) * (1.0 / _ROWS)

        @pl.when(i == 0)
        def _zero():
            out_ref[0, 0] = 0.0

        out_ref[0, 0] += partial


def kernel(input, target):
    tgt = target.astype(jnp.int32)
    tgt_safe = jnp.clip(tgt, 0, _COLS - 1)
    flat_idx = jnp.arange(_ROWS, dtype=jnp.int32) * _COLS + tgt_safe
    row_idx = flat_idx // _LANES
    lane = flat_idx % _LANES

    table = input.reshape(_TABLE_ROWS, _LANES)
    gathered = _sc_gather(table, row_idx)

    out = pl.pallas_call(
        _focal_kernel,
        grid=(_ROWS // _RBLK, _NCBLK),
        in_specs=[
            pl.BlockSpec((_RBLK, _CBLK), lambda i, j: (i, j)),
            pl.BlockSpec((_RBLK, 1), lambda i, j: (i, 0)),
            pl.BlockSpec((_RBLK, 1), lambda i, j: (i, 0)),
            pl.BlockSpec((_RBLK, _LANES), lambda i, j: (i, 0)),
        ],
        out_specs=pl.BlockSpec(
            (1, 1), lambda i, j: (0, 0), memory_space=pltpu.SMEM
        ),
        out_shape=jax.ShapeDtypeStruct((1, 1), jnp.float32),
        scratch_shapes=[
            pltpu.VMEM((_RBLK, 1), jnp.float32),
            pltpu.VMEM((_RBLK, 1), jnp.float32),
        ],
    )(input, tgt.reshape(_ROWS, 1), lane.reshape(_ROWS, 1), gathered)
    return out[0, 0]


# speedup vs baseline: 1.7075x; 1.7075x over previous
import jax
import jax.numpy as jnp
from jax.experimental import pallas as pl
from jax.experimental.pallas import tpu as pltpu

_ALPHA = 0.25
_IGNORE_INDEX = -100

_ROWS = 1024
_COLS = 100000
_RBLK = 256
_CBLK = 2048
_NCBLK = (_COLS + _CBLK - 1) // _CBLK
_CHUNKS = _CBLK // 128


def _focal_kernel(x_ref, tgt_ref, out_ref, m_ref, s_ref, t_ref):
    i = pl.program_id(0)
    j = pl.program_id(1)

    @pl.when(j == 0)
    def _init():
        m_ref[...] = jnp.full((_RBLK, 128), -jnp.inf, jnp.float32)
        s_ref[...] = jnp.zeros((_RBLK, 128), jnp.float32)
        t_ref[...] = jnp.zeros((_RBLK, 128), jnp.float32)

    x3 = x_ref[...].reshape(_RBLK, _CHUNKS, 128)
    tgt = tgt_ref[...]
    col = (
        j * _CBLK
        + jax.lax.broadcasted_iota(jnp.int32, (_RBLK, _CHUNKS, 128), 1) * 128
        + jax.lax.broadcasted_iota(jnp.int32, (_RBLK, _CHUNKS, 128), 2)
    )

    t_ref[...] += jnp.sum(
        jnp.where(col == tgt[:, :, None], x3, 0.0), axis=1
    )

    is_last = j == _NCBLK - 1

    @pl.when(jnp.logical_not(is_last))
    def _full_block():
        m_old = m_ref[...]
        m_new = jnp.maximum(m_old, jnp.max(x3, axis=1))
        s_ref[...] = s_ref[...] * jnp.exp(m_old - m_new) + jnp.sum(
            jnp.exp(x3 - m_new[:, None, :]), axis=1
        )
        m_ref[...] = m_new

    @pl.when(is_last)
    def _last_block():
        x3m = jnp.where(col < _COLS, x3, -jnp.inf)
        m_old = m_ref[...]
        m_lane = jnp.maximum(m_old, jnp.max(x3m, axis=1))
        s_lane = s_ref[...] * jnp.exp(m_old - m_lane) + jnp.sum(
            jnp.exp(x3m - m_lane[:, None, :]), axis=1
        )
        m_row = jnp.max(m_lane, axis=1, keepdims=True)
        s_row = jnp.sum(s_lane * jnp.exp(m_lane - m_row), axis=1, keepdims=True)
        t_row = jnp.sum(t_ref[...], axis=1, keepdims=True)
        nll = m_row + jnp.log(s_row) - t_row
        loss = jnp.where(tgt == _IGNORE_INDEX, 0.0, nll)
        pt = jnp.exp(-loss)
        fl = _ALPHA * (1.0 - pt) * (1.0 - pt) * loss
        partial = jnp.sum(fl) * (1.0 / _ROWS)

        @pl.when(i == 0)
        def _zero():
            out_ref[0, 0] = 0.0

        out_ref[0, 0] += partial


def kernel(input, target):
    tgt2d = target.astype(jnp.int32).reshape(_ROWS, 1)
    out = pl.pallas_call(
        _focal_kernel,
        grid=(_ROWS // _RBLK, _NCBLK),
        in_specs=[
            pl.BlockSpec((_RBLK, _CBLK), lambda i, j: (i, j)),
            pl.BlockSpec((_RBLK, 1), lambda i, j: (i, 0)),
        ],
        out_specs=pl.BlockSpec(
            (1, 1), lambda i, j: (0, 0), memory_space=pltpu.SMEM
        ),
        out_shape=jax.ShapeDtypeStruct((1, 1), jnp.float32),
        scratch_shapes=[
            pltpu.VMEM((_RBLK, 128), jnp.float32),
            pltpu.VMEM((_RBLK, 128), jnp.float32),
            pltpu.VMEM((_RBLK, 128), jnp.float32),
        ],
    )(input, tgt2d)
    return out[0, 0]


# speedup vs baseline: 2.0956x; 1.2273x over previous
import jax
import jax.numpy as jnp
from jax.experimental import pallas as pl
from jax.experimental.pallas import tpu as pltpu

_ALPHA = 0.25
_IGNORE_INDEX = -100

_ROWS = 1024
_COLS = 100000
_RBLK = 256
_NSPLIT = 4
_CSUB = 1024
_CHUNKS = _CSUB // 128
_CSTEP = _NSPLIT * _CSUB
_NJ = _COLS // _CSTEP + 1
_NCOLBLK = (_COLS + _CSUB - 1) // _CSUB

_LOG2E = 1.4426950408889634
_LN2 = 0.6931471805599453


def _focal_kernel(*refs):
    x_refs = refs[:_NSPLIT]
    tgt_ref, out_ref, m_ref, s_ref, t_ref = refs[_NSPLIT:]
    i = pl.program_id(0)
    j = pl.program_id(1)

    @pl.when(j == 0)
    def _init():
        m_ref[...] = jnp.full((_RBLK, 128), -jnp.inf, jnp.float32)
        s_ref[...] = jnp.zeros((_RBLK, 128), jnp.float32)
        t_ref[...] = jnp.zeros((_RBLK, 128), jnp.float32)

    tgt = tgt_ref[...]
    lane = jax.lax.broadcasted_iota(jnp.int32, (_RBLK, 128), 1)
    jbase = j * _CSTEP

    def process(chunks):
        bm = None
        for k, c, masked in chunks:
            y = x_refs[k][:, c * 128:(c + 1) * 128] * _LOG2E
            if masked:
                col = jbase + (k * _CSUB + c * 128) + lane
                y = jnp.where(col < _COLS, y, -jnp.inf)
            bm = y if bm is None else jnp.maximum(bm, y)
        m_old = m_ref[...]
        m_new = jnp.maximum(m_old, bm)
        s = s_ref[...] * jnp.exp2(m_old - m_new)
        t = t_ref[...]
        for k, c, masked in chunks:
            y = x_refs[k][:, c * 128:(c + 1) * 128] * _LOG2E
            col = jbase + (k * _CSUB + c * 128) + lane
            if masked:
                y = jnp.where(col < _COLS, y, -jnp.inf)
            s = s + jnp.exp2(y - m_new)
            t = t + jnp.where(col == tgt, y, 0.0)
        m_ref[...] = m_new
        s_ref[...] = s
        t_ref[...] = t
        return m_new, s, t

    is_last = j == _NJ - 1

    @pl.when(jnp.logical_not(is_last))
    def _full_step():
        process([(k, c, False) for k in range(_NSPLIT) for c in range(_CHUNKS)])

    @pl.when(is_last)
    def _last_step():
        base = (_NJ - 1) * _CSTEP
        chunks = []
        for k in range(_NSPLIT):
            for c in range(_CHUNKS):
                start = base + k * _CSUB + c * 128
                if start + 128 <= _COLS:
                    chunks.append((k, c, False))
                elif start < _COLS:
                    chunks.append((k, c, True))
        m_lane, s_lane, t_lane = process(chunks)
        m_row = jnp.max(m_lane, axis=1, keepdims=True)
        s_row = jnp.sum(
            s_lane * jnp.exp2(m_lane - m_row), axis=1, keepdims=True
        )
        t_row = jnp.sum(t_lane, axis=1, keepdims=True)
        nll = _LN2 * (m_row + jnp.log2(s_row) - t_row)
        loss = jnp.where(tgt == _IGNORE_INDEX, 0.0, nll)
        pt = jnp.exp(-loss)
        fl = _ALPHA * (1.0 - pt) * (1.0 - pt) * loss
        partial = jnp.sum(fl) * (1.0 / _ROWS)

        @pl.when(i == 0)
        def _zero():
            out_ref[0, 0] = 0.0

        out_ref[0, 0] += partial


def _make_index_map(k):
    def index_map(i, j):
        return (i, jnp.minimum(j * _NSPLIT + k, _NCOLBLK - 1))

    return index_map


def kernel(input, target):
    tgt2d = target.astype(jnp.int32).reshape(_ROWS, 1)
    out = pl.pallas_call(
        _focal_kernel,
        grid=(_ROWS // _RBLK, _NJ),
        in_specs=[
            pl.BlockSpec((_RBLK, _CSUB), _make_index_map(k))
            for k in range(_NSPLIT)
        ]
        + [pl.BlockSpec((_RBLK, 1), lambda i, j: (i, 0))],
        out_specs=pl.BlockSpec(
            (1, 1), lambda i, j: (0, 0), memory_space=pltpu.SMEM
        ),
        out_shape=jax.ShapeDtypeStruct((1, 1), jnp.float32),
        scratch_shapes=[
            pltpu.VMEM((_RBLK, 128), jnp.float32),
            pltpu.VMEM((_RBLK, 128), jnp.float32),
            pltpu.VMEM((_RBLK, 128), jnp.float32),
        ],
    )(*([input] * _NSPLIT), tgt2d)
    return out[0, 0]
